# Initial kernel scaffold; baseline (speedup 1.0000x reference)
import jax
import jax.numpy as jnp
from jax.experimental import pallas as pl


def kernel(input_ids, token_type_ids, word_emb, pos_emb, type_emb, gamma, beta):
    raise NotImplementedError("write your pallas kernel here")


# speedup vs baseline: 3.6717x; 3.6717x over previous
import functools

import jax
import jax.numpy as jnp
from jax import lax
from jax.experimental import pallas as pl
from jax.experimental.pallas import tpu as pltpu
from jax.experimental.pallas import tpu_sc as plsc

NC = 2
NS = 16
NW = NC * NS
LANES = 16
EPS = 1e-12


def _rsqrt(x):
    xh = x * 0.5
    i = plsc.bitcast(x, jnp.int32)
    i = jnp.int32(0x5F3759DF) - lax.shift_right_logical(i, 1)
    y = plsc.bitcast(i, jnp.float32)
    for _ in range(3):
        y = y * (1.5 - xh * y * y)
    return y


def _make_kernel(n_chunks, chunk, seq, hidden):
    per_w = n_chunks // NW
    nj = hidden // LANES
    mesh = plsc.VectorSubcoreMesh(core_axis_name="c", subcore_axis_name="s")

    def body(ids_ref, tid_ref, word_ref, pos_ref, te_ref, g_ref, b_ref,
             out_ref, idx_v, tid_v, rows_v, pos_v, te_v, g_v, b_v, sem):
        wid = lax.axis_index("s") * NC + lax.axis_index("c")
        pltpu.sync_copy(pos_ref.at[pl.ds(0, seq)], pos_v)
        pltpu.sync_copy(te_ref, te_v)
        pltpu.sync_copy(g_ref, g_v)
        pltpu.sync_copy(b_ref, b_v)

        g = [g_v[pl.ds(LANES * j, LANES)] for j in range(nj)]
        b = [b_v[pl.ds(LANES * j, LANES)] for j in range(nj)]
        t0 = [te_v[0, pl.ds(LANES * j, LANES)] for j in range(nj)]
        t1 = [te_v[1, pl.ds(LANES * j, LANES)] for j in range(nj)]

        def chunk_body(c, carry):
            r = wid * per_w + c
            s0 = (r % (seq // chunk)) * chunk
            pltpu.sync_copy(ids_ref.at[r], idx_v)
            pltpu.sync_copy(tid_ref.at[r], tid_v)
            pltpu.async_copy(word_ref.at[idx_v], rows_v, sem).wait()

            def tok_body(i, tcarry):
                tsplat = plsc.load_gather(
                    tid_v, [jnp.full((LANES,), i, jnp.int32)])
                pred = tsplat != 0
                x = []
                for j in range(nj):
                    w = rows_v[i, pl.ds(LANES * j, LANES)]
                    p = pos_v[s0 + i, pl.ds(LANES * j, LANES)]
                    t = jnp.where(pred, t1[j], t0[j])
                    x.append(w + p + t)
                s01 = x[0] + x[1]
                s23 = x[2] + x[3]
                s45 = x[4] + x[5]
                s67 = x[6] + x[7]
                svec = (s01 + s23) + (s45 + s67)
                q = [xj * xj for xj in x]
                q01 = q[0] + q[1]
                q23 = q[2] + q[3]
                q45 = q[4] + q[5]
                q67 = q[6] + q[7]
                qvec = (q01 + q23) + (q45 + q67)
                mean = jnp.broadcast_to(jnp.sum(svec), (LANES,)) * (1.0 / hidden)
                msq = jnp.broadcast_to(jnp.sum(qvec), (LANES,)) * (1.0 / hidden)
                var = msq - mean * mean
                inv = _rsqrt(var + EPS)
                for j in range(nj):
                    rows_v[i, pl.ds(LANES * j, LANES)] = (
                        (x[j] - mean) * inv * g[j] + b[j])
                return tcarry

            lax.fori_loop(0, chunk, tok_body, 0)
            pltpu.sync_copy(rows_v, out_ref.at[r])
            return carry

        lax.fori_loop(0, per_w, chunk_body, 0)

    kern = pl.kernel(
        body,
        out_type=jax.ShapeDtypeStruct((n_chunks, chunk, hidden), jnp.float32),
        mesh=mesh,
        compiler_params=pltpu.CompilerParams(needs_layout_passes=False),
        scratch_types=[
            pltpu.VMEM((chunk,), jnp.int32),
            pltpu.VMEM((chunk,), jnp.int32),
            pltpu.VMEM((chunk, hidden), jnp.float32),
            pltpu.VMEM((seq, hidden), jnp.float32),
            pltpu.VMEM((2, hidden), jnp.float32),
            pltpu.VMEM((hidden,), jnp.float32),
            pltpu.VMEM((hidden,), jnp.float32),
            pltpu.SemaphoreType.DMA,
        ],
    )
    return kern


@jax.jit
def kernel(input_ids, token_type_ids, word_emb, pos_emb, type_emb, gamma, beta):
    batch, seq = input_ids.shape
    hidden = word_emb.shape[1]
    chunk = seq // 2
    n_chunks = (batch * seq) // chunk
    ids2 = input_ids.reshape(n_chunks, chunk).astype(jnp.int32)
    tids2 = token_type_ids.reshape(n_chunks, chunk).astype(jnp.int32)
    kern = _make_kernel(n_chunks, chunk, seq, hidden)
    out = kern(ids2, tids2, word_emb, pos_emb, type_emb, gamma, beta)
    return out.reshape(batch, seq, hidden)


# speedup vs baseline: 5.4180x; 1.4756x over previous
import jax
import jax.numpy as jnp
from jax import lax
from jax.experimental import pallas as pl
from jax.experimental.pallas import tpu as pltpu
from jax.experimental.pallas import tpu_sc as plsc

NC = 2
NS = 16
NW = NC * NS
LANES = 16
NBUF = 4
EPS = 1e-12


def _rsqrt(x):
    xh = x * 0.5
    i = plsc.bitcast(x, jnp.int32)
    i = jnp.int32(0x5F3759DF) - lax.shift_right_logical(i, 1)
    y = plsc.bitcast(i, jnp.float32)
    for _ in range(2):
        y = y * (1.5 - xh * y * y)
    return y


def _make_kernel(n_chunks, chunk, seq, hidden):
    per_w = n_chunks // NW
    nj = hidden // LANES
    assert per_w % NBUF == 0 and per_w >= NBUF
    assert seq == 2 * chunk
    mesh = plsc.VectorSubcoreMesh(core_axis_name="c", subcore_axis_name="s")

    def body(ids_ref, tid_ref, word_ref, pos_ref, te_ref, g_ref, b_ref,
             out_ref, idx_all, tid_all, bufs, pos_v, te_v, g_v, b_v,
             gsems, osems):
        wid = lax.axis_index("s") * NC + lax.axis_index("c")
        base = wid * per_w
        pltpu.sync_copy(pos_ref.at[pl.ds(0, seq)], pos_v)
        pltpu.sync_copy(te_ref, te_v)
        pltpu.sync_copy(g_ref, g_v)
        pltpu.sync_copy(b_ref, b_v)
        pltpu.sync_copy(ids_ref.at[pl.ds(base, per_w)], idx_all)
        pltpu.sync_copy(tid_ref.at[pl.ds(base, per_w)], tid_all)

        g = [g_v[pl.ds(LANES * j, LANES)] for j in range(nj)]
        b = [b_v[pl.ds(LANES * j, LANES)] for j in range(nj)]
        t0 = [te_v[0, pl.ds(LANES * j, LANES)] for j in range(nj)]
        t1 = [te_v[1, pl.ds(LANES * j, LANES)] for j in range(nj)]

        def start_gather(c, bi):
            pltpu.async_copy(word_ref.at[idx_all.at[c]], bufs.at[bi],
                             gsems.at[bi])

        def wait_gather(c, bi):
            pltpu.make_async_copy(word_ref.at[idx_all.at[c]], bufs.at[bi],
                                  gsems.at[bi]).wait()

        def wait_out(bi):
            pltpu.make_async_copy(bufs.at[bi], out_ref.at[base],
                                  osems.at[bi]).wait()

        def compute(c, bi, s0):
            rows_v = bufs.at[bi]
            tid_c = tid_all.at[c]

            def tok_body(i, tcarry):
                tsplat = plsc.load_gather(
                    tid_c, [jnp.full((LANES,), i, jnp.int32)])
                pred = tsplat != 0
                x = []
                for j in range(nj):
                    w = rows_v[i, pl.ds(LANES * j, LANES)]
                    p = pos_v[s0 + i, pl.ds(LANES * j, LANES)]
                    t = jnp.where(pred, t1[j], t0[j])
                    x.append(w + p + t)
                svec = ((x[0] + x[1]) + (x[2] + x[3])) + (
                    (x[4] + x[5]) + (x[6] + x[7]))
                q = [xj * xj for xj in x]
                qvec = ((q[0] + q[1]) + (q[2] + q[3])) + (
                    (q[4] + q[5]) + (q[6] + q[7]))
                mean = jnp.broadcast_to(jnp.sum(svec), (LANES,)) * (1.0 / hidden)
                msq = jnp.broadcast_to(jnp.sum(qvec), (LANES,)) * (1.0 / hidden)
                var = msq - mean * mean
                inv = _rsqrt(var + EPS)
                for j in range(nj):
                    rows_v[i, pl.ds(LANES * j, LANES)] = (
                        (x[j] - mean) * inv * g[j] + b[j])
                return tcarry

            lax.fori_loop(0, chunk, tok_body, 0)

        for bi in range(NBUF - 1):
            start_gather(bi, bi)

        def ring_body(k, carry):
            for bi in range(NBUF):
                c = k * NBUF + bi
                wait_gather(c, bi)
                compute(c, bi, (bi & 1) * chunk)
                pltpu.async_copy(bufs.at[bi], out_ref.at[base + c],
                                 osems.at[bi])
                nb = (bi + NBUF - 1) % NBUF

                @pl.when(c + NBUF - 1 < per_w)
                def _issue():
                    if bi == 0:
                        @pl.when(k > 0)
                        def _():
                            wait_out(nb)
                    else:
                        wait_out(nb)
                    start_gather(c + NBUF - 1, nb)
            return carry

        lax.fori_loop(0, per_w // NBUF, ring_body, 0)
        for bi in range(NBUF):
            wait_out(bi)

    kern = pl.kernel(
        body,
        out_type=jax.ShapeDtypeStruct((n_chunks, chunk, hidden), jnp.float32),
        mesh=mesh,
        compiler_params=pltpu.CompilerParams(needs_layout_passes=False),
        scratch_types=[
            pltpu.VMEM((per_w, chunk), jnp.int32),
            pltpu.VMEM((per_w, chunk), jnp.int32),
            pltpu.VMEM((NBUF, chunk, hidden), jnp.float32),
            pltpu.VMEM((seq, hidden), jnp.float32),
            pltpu.VMEM((2, hidden), jnp.float32),
            pltpu.VMEM((hidden,), jnp.float32),
            pltpu.VMEM((hidden,), jnp.float32),
            pltpu.SemaphoreType.DMA((NBUF,)),
            pltpu.SemaphoreType.DMA((NBUF,)),
        ],
    )
    return kern


@jax.jit
def kernel(input_ids, token_type_ids, word_emb, pos_emb, type_emb, gamma, beta):
    batch, seq = input_ids.shape
    hidden = word_emb.shape[1]
    chunk = seq // 2
    n_chunks = (batch * seq) // chunk
    ids2 = input_ids.reshape(n_chunks, chunk).astype(jnp.int32)
    tids2 = token_type_ids.reshape(n_chunks, chunk).astype(jnp.int32)
    kern = _make_kernel(n_chunks, chunk, seq, hidden)
    out = kern(ids2, tids2, word_emb, pos_emb, type_emb, gamma, beta)
    return out.reshape(batch, seq, hidden)
